# baseline (speedup 1.0000x reference)
import functools

import jax
import jax.numpy as jnp
from jax import lax
from jax.experimental import pallas as pl
from jax.experimental.pallas import tpu as pltpu
from jax.experimental.pallas import tpu_sc as plsc

MARGIN = 0.2
NUM_CORES = 2
NUM_SUBCORES = 16
LANES = 16
NUM_WORKERS = NUM_CORES * NUM_SUBCORES

B, S, C = 128, 1024, 64
B_SC = 32
B_TC = B - B_SC
CHUNK_ROWS = 256
S_CHUNKS = S // CHUNK_ROWS
SC_CHUNKS_PER_WORKER = B_SC * S_CHUNKS // NUM_WORKERS
GROUPS = CHUNK_ROWS // LANES


def _make_sc_kernel():
    mesh = plsc.VectorSubcoreMesh(
        core_axis_name="c", subcore_axis_name="s",
        num_cores=NUM_CORES, num_subcores=NUM_SUBCORES)

    @functools.partial(
        pl.kernel,
        out_type=[
            jax.ShapeDtypeStruct((B_SC, S), jnp.float32),
            jax.ShapeDtypeStruct((B_SC, S), jnp.int32),
            jax.ShapeDtypeStruct((B_SC, S), jnp.int32),
            jax.ShapeDtypeStruct((NUM_WORKERS, LANES), jnp.float32),
            jax.ShapeDtypeStruct((NUM_WORKERS, LANES), jnp.float32),
        ],
        mesh=mesh,
        compiler_params=pltpu.CompilerParams(needs_layout_passes=False),
        scratch_types=[
            pltpu.VMEM((C, CHUNK_ROWS), jnp.float32),
            pltpu.VMEM((C, CHUNK_ROWS), jnp.int32),
            pltpu.VMEM((CHUNK_ROWS,), jnp.int32),
            pltpu.VMEM((CHUNK_ROWS,), jnp.float32),
            pltpu.VMEM((CHUNK_ROWS,), jnp.int32),
            pltpu.VMEM((CHUNK_ROWS,), jnp.int32),
            pltpu.VMEM((LANES,), jnp.float32),
        ],
    )
    def sc_kernel(scores_hbm, ents_hbm, gold_hbm,
                  maxs_hbm, pred_hbm, maski_hbm, lsum_hbm, lcnt_hbm,
                  s_v, e_v, g_v, mx_v, pd_v, mk_v, acc_v):
        wid = lax.axis_index("s") * NUM_CORES + lax.axis_index("c")
        zf = jnp.zeros((LANES,), jnp.float32)
        zi = jnp.zeros((LANES,), jnp.int32)

        def chunk_body(ci, carry):
            la, ca = carry
            ck = wid * SC_CHUNKS_PER_WORKER + ci
            b = ck // S_CHUNKS
            s0 = (ck % S_CHUNKS) * CHUNK_ROWS
            pltpu.sync_copy(
                scores_hbm.at[b, :, pl.ds(s0, CHUNK_ROWS)], s_v)
            pltpu.sync_copy(
                ents_hbm.at[b, :, pl.ds(s0, CHUNK_ROWS)], e_v)
            pltpu.sync_copy(gold_hbm.at[b, pl.ds(s0, CHUNK_ROWS)], g_v)

            def group_body(gi, carry2):
                la, ca = carry2
                goldv = g_v[pl.ds(gi * LANES, LANES)]
                rmax = jnp.full((LANES,), -3.4e38, jnp.float32)
                pred = zi
                for c in range(C):
                    vs = s_v[c, pl.ds(gi * LANES, LANES)]
                    ve = e_v[c, pl.ds(gi * LANES, LANES)]
                    pos = ve == goldv
                    elem = jnp.maximum(
                        jnp.where(pos, MARGIN - vs, MARGIN + vs), 0.0)
                    maskb = ve > 0
                    la = la + jnp.where(maskb, elem, zf)
                    ca = ca + plsc.all_reduce_population_count(maskb)
                    takes = vs > rmax
                    rmax = jnp.maximum(rmax, vs)
                    pred = jnp.where(takes, ve, pred)
                above = rmax > 0.0
                predz = jnp.where(above & (pred != 0), pred, zi)
                mx_v[pl.ds(gi * LANES, LANES)] = rmax
                pd_v[pl.ds(gi * LANES, LANES)] = predz
                mk_v[pl.ds(gi * LANES, LANES)] = jnp.where(above, 1, 0)
                return la, ca

            la, ca = lax.fori_loop(0, GROUPS, group_body, (la, ca))
            pltpu.sync_copy(mx_v, maxs_hbm.at[b, pl.ds(s0, CHUNK_ROWS)])
            pltpu.sync_copy(pd_v, pred_hbm.at[b, pl.ds(s0, CHUNK_ROWS)])
            pltpu.sync_copy(mk_v, maski_hbm.at[b, pl.ds(s0, CHUNK_ROWS)])
            return la, ca

        la, ca = lax.fori_loop(0, SC_CHUNKS_PER_WORKER, chunk_body, (zf, zi))
        acc_v[...] = la
        pltpu.sync_copy(acc_v, lsum_hbm.at[wid])
        acc_v[...] = ca.astype(jnp.float32) * 0.0625
        pltpu.sync_copy(acc_v, lcnt_hbm.at[wid])

    return sc_kernel


S_BLK = 256
N_SJ = S // S_BLK


def _tc_body(s_ref, e_ref, g_ref, mx_ref, pd_ref, mk_ref, ls_ref, lc_ref):
    vs = s_ref[0]
    ve = e_ref[0]
    gold = g_ref[0]
    pos = ve == gold
    maskb = ve > 0
    elem = jnp.maximum(jnp.where(pos, MARGIN - vs, MARGIN + vs), 0.0)
    elemz = jnp.where(maskb, elem, 0.0)
    ls_row = jnp.sum(elemz, axis=0, keepdims=True)
    lc_row = jnp.sum(maskb.astype(jnp.float32), axis=0, keepdims=True)
    first_b = pl.program_id(1) == 0

    @pl.when(first_b)
    def _():
        ls_ref[0] = ls_row
        lc_ref[0] = lc_row

    @pl.when(jnp.logical_not(first_b))
    def _():
        ls_ref[0] += ls_row
        lc_ref[0] += lc_row

    rmax = jnp.max(vs, axis=0, keepdims=True)
    ids = lax.broadcasted_iota(jnp.int32, (C, S_BLK), 0)
    cand = jnp.where(vs == rmax, ids, C)
    aidx = jnp.min(cand, axis=0, keepdims=True)
    pred = jnp.sum(jnp.where(ids == aidx, ve, 0), axis=0, keepdims=True)
    above = rmax > 0.0
    predz = jnp.where(above & (pred != 0), pred, 0)
    mx_ref[0] = rmax
    pd_ref[0] = predz
    mk_ref[0] = jnp.where(above, 1, 0)


def _tc_call(scores_t, ents_t, gold3):
    blk_in = pl.BlockSpec((1, C, S_BLK), lambda sj, b: (B_SC + b, 0, sj))
    blk_g = pl.BlockSpec((1, 1, S_BLK), lambda sj, b: (B_SC + b, 0, sj))
    blk_out = pl.BlockSpec((1, 1, S_BLK), lambda sj, b: (b, 0, sj))
    blk_acc = pl.BlockSpec((1, 1, S_BLK), lambda sj, b: (0, 0, sj))
    out_shape = jax.ShapeDtypeStruct((B_TC, 1, S), jnp.float32)
    out_shape_i = jax.ShapeDtypeStruct((B_TC, 1, S), jnp.int32)
    acc_shape = jax.ShapeDtypeStruct((1, 1, S), jnp.float32)
    return pl.pallas_call(
        _tc_body,
        grid=(N_SJ, B_TC),
        in_specs=[blk_in, blk_in, blk_g],
        out_specs=[blk_out, blk_out, blk_out, blk_acc, blk_acc],
        out_shape=[out_shape, out_shape_i, out_shape_i, acc_shape, acc_shape],
        compiler_params=pltpu.CompilerParams(
            dimension_semantics=("arbitrary", "arbitrary")),
    )(scores_t, ents_t, gold3)


def kernel(linking_scores, candidate_spans, candidate_entities, gold_entities):
    del candidate_spans
    scores_t = linking_scores.transpose(0, 2, 1)
    ents_t = candidate_entities.astype(jnp.int32).transpose(0, 2, 1)
    gold = gold_entities.reshape(B, S).astype(jnp.int32)
    gold3 = gold.reshape(B, 1, S)
    sc_maxs, sc_pred, sc_maski, sc_ls, sc_lc = _make_sc_kernel()(
        scores_t, ents_t, gold)
    tc_mx, tc_pd, tc_mk, tc_ls, tc_lc = _tc_call(scores_t, ents_t, gold3)
    maxs = jnp.concatenate([sc_maxs, tc_mx.reshape(B_TC, S)], axis=0)
    pred = jnp.concatenate([sc_pred, tc_pd.reshape(B_TC, S)], axis=0)
    maski = jnp.concatenate([sc_maski, tc_mk.reshape(B_TC, S)], axis=0)
    lsum = jnp.sum(sc_ls) + jnp.sum(tc_ls.reshape(S))
    lcnt = jnp.sum(sc_lc) + jnp.sum(tc_lc.reshape(S))
    loss = lsum / jnp.maximum(lcnt, 1.0)
    return (
        loss.astype(linking_scores.dtype),
        maxs,
        pred.astype(candidate_entities.dtype),
        maski.astype(jnp.bool_),
    )


# speedup vs baseline: 1.0018x; 1.0018x over previous
import functools

import jax
import jax.numpy as jnp
from jax import lax
from jax.experimental import pallas as pl
from jax.experimental.pallas import tpu as pltpu
from jax.experimental.pallas import tpu_sc as plsc

MARGIN = 0.2
NUM_CORES = 2
NUM_SUBCORES = 16
LANES = 16
NUM_WORKERS = NUM_CORES * NUM_SUBCORES

B, S, C = 128, 1024, 64
B_SC = 32
B_TC = B - B_SC
CHUNK_ROWS = 256
S_CHUNKS = S // CHUNK_ROWS
SC_CHUNKS_PER_WORKER = B_SC * S_CHUNKS // NUM_WORKERS
GROUPS = CHUNK_ROWS // LANES


def _make_sc_kernel():
    mesh = plsc.VectorSubcoreMesh(
        core_axis_name="c", subcore_axis_name="s",
        num_cores=NUM_CORES, num_subcores=NUM_SUBCORES)

    @functools.partial(
        pl.kernel,
        out_type=[
            jax.ShapeDtypeStruct((B_SC, S), jnp.float32),
            jax.ShapeDtypeStruct((B_SC, S), jnp.int32),
            jax.ShapeDtypeStruct((B_SC, S), jnp.int32),
            jax.ShapeDtypeStruct((NUM_WORKERS, LANES), jnp.float32),
            jax.ShapeDtypeStruct((NUM_WORKERS, LANES), jnp.float32),
        ],
        mesh=mesh,
        compiler_params=pltpu.CompilerParams(needs_layout_passes=False),
        scratch_types=[
            pltpu.VMEM((C, CHUNK_ROWS), jnp.float32),
            pltpu.VMEM((C, CHUNK_ROWS), jnp.int32),
            pltpu.VMEM((CHUNK_ROWS,), jnp.int32),
            pltpu.VMEM((CHUNK_ROWS,), jnp.float32),
            pltpu.VMEM((CHUNK_ROWS,), jnp.int32),
            pltpu.VMEM((CHUNK_ROWS,), jnp.int32),
            pltpu.VMEM((LANES,), jnp.float32),
        ],
    )
    def sc_kernel(scores_hbm, ents_hbm, gold_hbm,
                  maxs_hbm, pred_hbm, maski_hbm, lsum_hbm, lcnt_hbm,
                  s_v, e_v, g_v, mx_v, pd_v, mk_v, acc_v):
        wid = lax.axis_index("s") * NUM_CORES + lax.axis_index("c")
        zf = jnp.zeros((LANES,), jnp.float32)
        zi = jnp.zeros((LANES,), jnp.int32)

        def chunk_body(ci, carry):
            la, ca = carry
            ck = wid * SC_CHUNKS_PER_WORKER + ci
            b = ck // S_CHUNKS
            s0 = (ck % S_CHUNKS) * CHUNK_ROWS
            pltpu.sync_copy(
                scores_hbm.at[b, :, pl.ds(s0, CHUNK_ROWS)], s_v)
            pltpu.sync_copy(
                ents_hbm.at[b, :, pl.ds(s0, CHUNK_ROWS)], e_v)
            pltpu.sync_copy(gold_hbm.at[b, pl.ds(s0, CHUNK_ROWS)], g_v)

            def group_body(gi, carry2):
                la, ca = carry2
                goldv = g_v[pl.ds(gi * LANES, LANES)]
                rmax = jnp.full((LANES,), -3.4e38, jnp.float32)
                pred = zi
                for c in range(C):
                    vs = s_v[c, pl.ds(gi * LANES, LANES)]
                    ve = e_v[c, pl.ds(gi * LANES, LANES)]
                    pos = ve == goldv
                    elem = jnp.maximum(
                        jnp.where(pos, MARGIN - vs, MARGIN + vs), 0.0)
                    maskb = ve > 0
                    la = la + jnp.where(maskb, elem, zf)
                    ca = ca + plsc.all_reduce_population_count(maskb)
                    takes = vs > rmax
                    rmax = jnp.maximum(rmax, vs)
                    pred = jnp.where(takes, ve, pred)
                above = rmax > 0.0
                predz = jnp.where(above & (pred != 0), pred, zi)
                mx_v[pl.ds(gi * LANES, LANES)] = rmax
                pd_v[pl.ds(gi * LANES, LANES)] = predz
                mk_v[pl.ds(gi * LANES, LANES)] = jnp.where(above, 1, 0)
                return la, ca

            la, ca = lax.fori_loop(0, GROUPS, group_body, (la, ca))
            pltpu.sync_copy(mx_v, maxs_hbm.at[b, pl.ds(s0, CHUNK_ROWS)])
            pltpu.sync_copy(pd_v, pred_hbm.at[b, pl.ds(s0, CHUNK_ROWS)])
            pltpu.sync_copy(mk_v, maski_hbm.at[b, pl.ds(s0, CHUNK_ROWS)])
            return la, ca

        la, ca = lax.fori_loop(0, SC_CHUNKS_PER_WORKER, chunk_body, (zf, zi))
        acc_v[...] = la
        pltpu.sync_copy(acc_v, lsum_hbm.at[wid])
        acc_v[...] = ca.astype(jnp.float32) * 0.0625
        pltpu.sync_copy(acc_v, lcnt_hbm.at[wid])

    return sc_kernel


S_BLK = 256
N_SJ = S // S_BLK


def _tc_body(s_ref, e_ref, g_ref, mx_ref, pd_ref, mk_ref, ls_ref, lc_ref):
    vs = s_ref[0]
    ve = e_ref[0]
    gold = g_ref[0]
    pos = ve == gold
    maskb = ve > 0
    elem = jnp.maximum(jnp.where(pos, MARGIN - vs, MARGIN + vs), 0.0)
    elemz = jnp.where(maskb, elem, 0.0)
    ls_ref[0] = jnp.sum(elemz, axis=0, keepdims=True)
    lc_ref[0] = jnp.sum(maskb.astype(jnp.float32), axis=0, keepdims=True)
    rmax = jnp.max(vs, axis=0, keepdims=True)
    ids = lax.broadcasted_iota(jnp.int32, (C, S_BLK), 0)
    cand = jnp.where(vs == rmax, ids, C)
    aidx = jnp.min(cand, axis=0, keepdims=True)
    pred = jnp.sum(jnp.where(ids == aidx, ve, 0), axis=0, keepdims=True)
    above = rmax > 0.0
    predz = jnp.where(above & (pred != 0), pred, 0)
    mx_ref[0] = rmax
    pd_ref[0] = predz
    mk_ref[0] = jnp.where(above, 1, 0)


def _tc_call(scores_t, ents_t, gold3):
    blk_in = pl.BlockSpec((1, C, S_BLK), lambda b, sj: (B_SC + b, 0, sj))
    blk_g = pl.BlockSpec((1, 1, S_BLK), lambda b, sj: (B_SC + b, 0, sj))
    blk_out = pl.BlockSpec((1, 1, S_BLK), lambda b, sj: (b, 0, sj))
    out_shape = jax.ShapeDtypeStruct((B_TC, 1, S), jnp.float32)
    out_shape_i = jax.ShapeDtypeStruct((B_TC, 1, S), jnp.int32)
    return pl.pallas_call(
        _tc_body,
        grid=(B_TC, N_SJ),
        in_specs=[blk_in, blk_in, blk_g],
        out_specs=[blk_out] * 5,
        out_shape=[out_shape, out_shape_i, out_shape_i, out_shape, out_shape],
        compiler_params=pltpu.CompilerParams(
            dimension_semantics=("arbitrary", "arbitrary")),
    )(scores_t, ents_t, gold3)


def kernel(linking_scores, candidate_spans, candidate_entities, gold_entities):
    del candidate_spans
    scores_t = linking_scores.transpose(0, 2, 1)
    ents_t = candidate_entities.astype(jnp.int32).transpose(0, 2, 1)
    gold = gold_entities.reshape(B, S).astype(jnp.int32)
    gold3 = gold.reshape(B, 1, S)
    sc_maxs, sc_pred, sc_maski, sc_ls, sc_lc = _make_sc_kernel()(
        scores_t, ents_t, gold)
    tc_mx, tc_pd, tc_mk, tc_ls, tc_lc = _tc_call(scores_t, ents_t, gold3)
    maxs = jnp.concatenate([sc_maxs, tc_mx.reshape(B_TC, S)], axis=0)
    pred = jnp.concatenate([sc_pred, tc_pd.reshape(B_TC, S)], axis=0)
    maski = jnp.concatenate([sc_maski, tc_mk.reshape(B_TC, S)], axis=0)
    lsum = jnp.sum(sc_ls) + jnp.sum(tc_ls)
    lcnt = jnp.sum(sc_lc) + jnp.sum(tc_lc)
    loss = lsum / jnp.maximum(lcnt, 1.0)
    return (
        loss.astype(linking_scores.dtype),
        maxs,
        pred.astype(candidate_entities.dtype),
        maski.astype(jnp.bool_),
    )


# speedup vs baseline: 2.6928x; 2.6879x over previous
import functools

import jax
import jax.numpy as jnp
from jax import lax
from jax.experimental import pallas as pl
from jax.experimental.pallas import tpu as pltpu
from jax.experimental.pallas import tpu_sc as plsc

MARGIN = 0.2
NUM_CORES = 2
NUM_SUBCORES = 16
LANES = 16
NUM_WORKERS = NUM_CORES * NUM_SUBCORES

B, S, C = 128, 1024, 64
B_SC = 56
B_TC = B - B_SC
CHUNK_ROWS = 256
S_CHUNKS = S // CHUNK_ROWS
SC_CHUNKS_PER_WORKER = B_SC * S_CHUNKS // NUM_WORKERS
GROUPS = CHUNK_ROWS // LANES


def _make_sc_kernel():
    mesh = plsc.VectorSubcoreMesh(
        core_axis_name="c", subcore_axis_name="s",
        num_cores=NUM_CORES, num_subcores=NUM_SUBCORES)

    @functools.partial(
        pl.kernel,
        out_type=[
            jax.ShapeDtypeStruct((B_SC, S), jnp.float32),
            jax.ShapeDtypeStruct((B_SC, S), jnp.int32),
            jax.ShapeDtypeStruct((B_SC, S), jnp.int32),
            jax.ShapeDtypeStruct((NUM_WORKERS, LANES), jnp.float32),
            jax.ShapeDtypeStruct((NUM_WORKERS, LANES), jnp.float32),
        ],
        mesh=mesh,
        compiler_params=pltpu.CompilerParams(needs_layout_passes=False),
        scratch_types=[
            pltpu.VMEM((C, CHUNK_ROWS), jnp.float32),
            pltpu.VMEM((C, CHUNK_ROWS), jnp.int32),
            pltpu.VMEM((CHUNK_ROWS,), jnp.int32),
            pltpu.VMEM((CHUNK_ROWS,), jnp.float32),
            pltpu.VMEM((CHUNK_ROWS,), jnp.int32),
            pltpu.VMEM((CHUNK_ROWS,), jnp.int32),
            pltpu.VMEM((LANES,), jnp.float32),
        ],
    )
    def sc_kernel(scores_hbm, ents_hbm, gold_hbm,
                  maxs_hbm, pred_hbm, maski_hbm, lsum_hbm, lcnt_hbm,
                  s_v, e_v, g_v, mx_v, pd_v, mk_v, acc_v):
        wid = lax.axis_index("s") * NUM_CORES + lax.axis_index("c")
        zf = jnp.zeros((LANES,), jnp.float32)
        zi = jnp.zeros((LANES,), jnp.int32)

        def chunk_body(ci, carry):
            la, ca = carry
            ck = wid * SC_CHUNKS_PER_WORKER + ci
            b = ck // S_CHUNKS
            s0 = (ck % S_CHUNKS) * CHUNK_ROWS
            pltpu.sync_copy(
                scores_hbm.at[b, :, pl.ds(s0, CHUNK_ROWS)], s_v)
            pltpu.sync_copy(
                ents_hbm.at[b, :, pl.ds(s0, CHUNK_ROWS)], e_v)
            pltpu.sync_copy(gold_hbm.at[b, pl.ds(s0, CHUNK_ROWS)], g_v)

            def group_body(gi, carry2):
                la, ca = carry2
                goldv = g_v[pl.ds(gi * LANES, LANES)]
                rmax = jnp.full((LANES,), -3.4e38, jnp.float32)
                pred = zi
                for c in range(C):
                    vs = s_v[c, pl.ds(gi * LANES, LANES)]
                    ve = e_v[c, pl.ds(gi * LANES, LANES)]
                    pos = ve == goldv
                    elem = jnp.maximum(
                        jnp.where(pos, MARGIN - vs, MARGIN + vs), 0.0)
                    maskb = ve > 0
                    la = la + jnp.where(maskb, elem, zf)
                    ca = ca + plsc.all_reduce_population_count(maskb)
                    takes = vs > rmax
                    rmax = jnp.maximum(rmax, vs)
                    pred = jnp.where(takes, ve, pred)
                above = rmax > 0.0
                predz = jnp.where(above & (pred != 0), pred, zi)
                mx_v[pl.ds(gi * LANES, LANES)] = rmax
                pd_v[pl.ds(gi * LANES, LANES)] = predz
                mk_v[pl.ds(gi * LANES, LANES)] = jnp.where(above, 1, 0)
                return la, ca

            la, ca = lax.fori_loop(0, GROUPS, group_body, (la, ca))
            pltpu.sync_copy(mx_v, maxs_hbm.at[b, pl.ds(s0, CHUNK_ROWS)])
            pltpu.sync_copy(pd_v, pred_hbm.at[b, pl.ds(s0, CHUNK_ROWS)])
            pltpu.sync_copy(mk_v, maski_hbm.at[b, pl.ds(s0, CHUNK_ROWS)])
            return la, ca

        la, ca = lax.fori_loop(0, SC_CHUNKS_PER_WORKER, chunk_body, (zf, zi))
        acc_v[...] = la
        pltpu.sync_copy(acc_v, lsum_hbm.at[wid])
        acc_v[...] = ca.astype(jnp.float32) * 0.0625
        pltpu.sync_copy(acc_v, lcnt_hbm.at[wid])

    return sc_kernel


def _tc_body(s_ref, e_ref, g_ref, mx_ref, pd_ref, mk_ref, ls_ref, lc_ref):
    vs = s_ref[0]
    ve = e_ref[0]
    gold = g_ref[0]
    pos = ve == gold
    maskb = ve > 0
    elem = jnp.maximum(jnp.where(pos, MARGIN - vs, MARGIN + vs), 0.0)
    elemz = jnp.where(maskb, elem, 0.0)
    ls_ref[0] = jnp.sum(elemz, axis=0, keepdims=True)
    lc_ref[0] = jnp.sum(maskb.astype(jnp.float32), axis=0, keepdims=True)
    rmax = jnp.max(vs, axis=0, keepdims=True)
    ids = lax.broadcasted_iota(jnp.int32, (C, S), 0)
    cand = jnp.where(vs == rmax, ids, C)
    aidx = jnp.min(cand, axis=0, keepdims=True)
    pred = jnp.sum(jnp.where(ids == aidx, ve, 0), axis=0, keepdims=True)
    above = rmax > 0.0
    predz = jnp.where(above & (pred != 0), pred, 0)
    mx_ref[0] = rmax
    pd_ref[0] = predz
    mk_ref[0] = jnp.where(above, 1, 0)


def _tc_call(scores_t, ents_t, gold3):
    blk_in = pl.BlockSpec((1, C, S), lambda b: (B_SC + b, 0, 0))
    blk_g = pl.BlockSpec((1, 1, S), lambda b: (B_SC + b, 0, 0))
    blk_out = pl.BlockSpec((1, 1, S), lambda b: (b, 0, 0))
    out_shape = jax.ShapeDtypeStruct((B_TC, 1, S), jnp.float32)
    out_shape_i = jax.ShapeDtypeStruct((B_TC, 1, S), jnp.int32)
    return pl.pallas_call(
        _tc_body,
        grid=(B_TC,),
        in_specs=[blk_in, blk_in, blk_g],
        out_specs=[blk_out] * 5,
        out_shape=[out_shape, out_shape_i, out_shape_i, out_shape, out_shape],
    )(scores_t, ents_t, gold3)


def kernel(linking_scores, candidate_spans, candidate_entities, gold_entities):
    del candidate_spans
    scores_t = linking_scores.transpose(0, 2, 1)
    ents_t = candidate_entities.astype(jnp.int32).transpose(0, 2, 1)
    gold = gold_entities.reshape(B, S).astype(jnp.int32)
    gold3 = gold.reshape(B, 1, S)
    sc_maxs, sc_pred, sc_maski, sc_ls, sc_lc = _make_sc_kernel()(
        scores_t, ents_t, gold)
    tc_mx, tc_pd, tc_mk, tc_ls, tc_lc = _tc_call(scores_t, ents_t, gold3)
    maxs = jnp.concatenate([sc_maxs, tc_mx.reshape(B_TC, S)], axis=0)
    pred = jnp.concatenate([sc_pred, tc_pd.reshape(B_TC, S)], axis=0)
    maski = jnp.concatenate([sc_maski, tc_mk.reshape(B_TC, S)], axis=0)
    lsum = jnp.sum(sc_ls) + jnp.sum(tc_ls)
    lcnt = jnp.sum(sc_lc) + jnp.sum(tc_lc)
    loss = lsum / jnp.maximum(lcnt, 1.0)
    return (
        loss.astype(linking_scores.dtype),
        maxs,
        pred.astype(candidate_entities.dtype),
        maski.astype(jnp.bool_),
    )


# speedup vs baseline: 2.8778x; 1.0687x over previous
import functools

import jax
import jax.numpy as jnp
from jax import lax
from jax.experimental import pallas as pl
from jax.experimental.pallas import tpu as pltpu
from jax.experimental.pallas import tpu_sc as plsc

MARGIN = 0.2
NUM_CORES = 2
NUM_SUBCORES = 16
LANES = 16
NUM_WORKERS = NUM_CORES * NUM_SUBCORES

B, S, C = 128, 1024, 64
B_SC = 64
B_TC = B - B_SC
CHUNK_ROWS = 256
S_CHUNKS = S // CHUNK_ROWS
SC_CHUNKS_PER_WORKER = B_SC * S_CHUNKS // NUM_WORKERS
GROUPS = CHUNK_ROWS // LANES


def _make_sc_kernel():
    mesh = plsc.VectorSubcoreMesh(
        core_axis_name="c", subcore_axis_name="s",
        num_cores=NUM_CORES, num_subcores=NUM_SUBCORES)

    @functools.partial(
        pl.kernel,
        out_type=[
            jax.ShapeDtypeStruct((B_SC, S), jnp.float32),
            jax.ShapeDtypeStruct((B_SC, S), jnp.int32),
            jax.ShapeDtypeStruct((B_SC, S), jnp.int32),
            jax.ShapeDtypeStruct((NUM_WORKERS, LANES), jnp.float32),
            jax.ShapeDtypeStruct((NUM_WORKERS, LANES), jnp.float32),
        ],
        mesh=mesh,
        compiler_params=pltpu.CompilerParams(needs_layout_passes=False),
        scratch_types=[
            pltpu.VMEM((C, CHUNK_ROWS), jnp.float32),
            pltpu.VMEM((C, CHUNK_ROWS), jnp.int32),
            pltpu.VMEM((CHUNK_ROWS,), jnp.int32),
            pltpu.VMEM((CHUNK_ROWS,), jnp.float32),
            pltpu.VMEM((CHUNK_ROWS,), jnp.int32),
            pltpu.VMEM((CHUNK_ROWS,), jnp.int32),
            pltpu.VMEM((LANES,), jnp.float32),
        ],
    )
    def sc_kernel(scores_hbm, ents_hbm, gold_hbm,
                  maxs_hbm, pred_hbm, maski_hbm, lsum_hbm, lcnt_hbm,
                  s_v, e_v, g_v, mx_v, pd_v, mk_v, acc_v):
        wid = lax.axis_index("s") * NUM_CORES + lax.axis_index("c")
        zf = jnp.zeros((LANES,), jnp.float32)
        zi = jnp.zeros((LANES,), jnp.int32)

        def chunk_body(ci, carry):
            la, ca = carry
            ck = wid * SC_CHUNKS_PER_WORKER + ci
            b = ck // S_CHUNKS
            s0 = (ck % S_CHUNKS) * CHUNK_ROWS
            pltpu.sync_copy(
                scores_hbm.at[b, :, pl.ds(s0, CHUNK_ROWS)], s_v)
            pltpu.sync_copy(
                ents_hbm.at[b, :, pl.ds(s0, CHUNK_ROWS)], e_v)
            pltpu.sync_copy(gold_hbm.at[b, pl.ds(s0, CHUNK_ROWS)], g_v)

            def group_body(gi, carry2):
                la, ca = carry2
                goldv = g_v[pl.ds(gi * LANES, LANES)]
                rmax = jnp.full((LANES,), -3.4e38, jnp.float32)
                pred = zi
                for c in range(C):
                    vs = s_v[c, pl.ds(gi * LANES, LANES)]
                    ve = e_v[c, pl.ds(gi * LANES, LANES)]
                    pos = ve == goldv
                    elem = jnp.maximum(
                        jnp.where(pos, MARGIN - vs, MARGIN + vs), 0.0)
                    maskb = ve > 0
                    la = la + jnp.where(maskb, elem, zf)
                    ca = ca + plsc.all_reduce_population_count(maskb)
                    takes = vs > rmax
                    rmax = jnp.maximum(rmax, vs)
                    pred = jnp.where(takes, ve, pred)
                above = rmax > 0.0
                predz = jnp.where(above & (pred != 0), pred, zi)
                mx_v[pl.ds(gi * LANES, LANES)] = rmax
                pd_v[pl.ds(gi * LANES, LANES)] = predz
                mk_v[pl.ds(gi * LANES, LANES)] = jnp.where(above, 1, 0)
                return la, ca

            la, ca = lax.fori_loop(0, GROUPS, group_body, (la, ca))
            pltpu.sync_copy(mx_v, maxs_hbm.at[b, pl.ds(s0, CHUNK_ROWS)])
            pltpu.sync_copy(pd_v, pred_hbm.at[b, pl.ds(s0, CHUNK_ROWS)])
            pltpu.sync_copy(mk_v, maski_hbm.at[b, pl.ds(s0, CHUNK_ROWS)])
            return la, ca

        la, ca = lax.fori_loop(0, SC_CHUNKS_PER_WORKER, chunk_body, (zf, zi))
        acc_v[...] = la
        pltpu.sync_copy(acc_v, lsum_hbm.at[wid])
        acc_v[...] = ca.astype(jnp.float32) * 0.0625
        pltpu.sync_copy(acc_v, lcnt_hbm.at[wid])

    return sc_kernel


def _tc_body(s_ref, e_ref, g_ref, mx_ref, pd_ref, mk_ref, ls_ref, lc_ref):
    vs = s_ref[0]
    ve = e_ref[0]
    gold = g_ref[0]
    pos = ve == gold
    maskb = ve > 0
    elem = jnp.maximum(jnp.where(pos, MARGIN - vs, MARGIN + vs), 0.0)
    elemz = jnp.where(maskb, elem, 0.0)
    ls_ref[0] = jnp.sum(elemz, axis=0, keepdims=True)
    lc_ref[0] = jnp.sum(maskb.astype(jnp.float32), axis=0, keepdims=True)
    rmax = jnp.max(vs, axis=0, keepdims=True)
    ids = lax.broadcasted_iota(jnp.int32, (C, S), 0)
    cand = jnp.where(vs == rmax, ids, C)
    aidx = jnp.min(cand, axis=0, keepdims=True)
    pred = jnp.sum(jnp.where(ids == aidx, ve, 0), axis=0, keepdims=True)
    above = rmax > 0.0
    predz = jnp.where(above & (pred != 0), pred, 0)
    mx_ref[0] = rmax
    pd_ref[0] = predz
    mk_ref[0] = jnp.where(above, 1, 0)


def _tc_call(scores_t, ents_t, gold3):
    blk_in = pl.BlockSpec((1, C, S), lambda b: (B_SC + b, 0, 0))
    blk_g = pl.BlockSpec((1, 1, S), lambda b: (B_SC + b, 0, 0))
    blk_out = pl.BlockSpec((1, 1, S), lambda b: (b, 0, 0))
    out_shape = jax.ShapeDtypeStruct((B_TC, 1, S), jnp.float32)
    out_shape_i = jax.ShapeDtypeStruct((B_TC, 1, S), jnp.int32)
    return pl.pallas_call(
        _tc_body,
        grid=(B_TC,),
        in_specs=[blk_in, blk_in, blk_g],
        out_specs=[blk_out] * 5,
        out_shape=[out_shape, out_shape_i, out_shape_i, out_shape, out_shape],
    )(scores_t, ents_t, gold3)


def kernel(linking_scores, candidate_spans, candidate_entities, gold_entities):
    del candidate_spans
    scores_t = linking_scores.transpose(0, 2, 1)
    ents_t = candidate_entities.astype(jnp.int32).transpose(0, 2, 1)
    gold = gold_entities.reshape(B, S).astype(jnp.int32)
    gold3 = gold.reshape(B, 1, S)
    sc_maxs, sc_pred, sc_maski, sc_ls, sc_lc = _make_sc_kernel()(
        scores_t, ents_t, gold)
    tc_mx, tc_pd, tc_mk, tc_ls, tc_lc = _tc_call(scores_t, ents_t, gold3)
    maxs = jnp.concatenate([sc_maxs, tc_mx.reshape(B_TC, S)], axis=0)
    pred = jnp.concatenate([sc_pred, tc_pd.reshape(B_TC, S)], axis=0)
    maski = jnp.concatenate([sc_maski, tc_mk.reshape(B_TC, S)], axis=0)
    lsum = jnp.sum(sc_ls) + jnp.sum(tc_ls)
    lcnt = jnp.sum(sc_lc) + jnp.sum(tc_lc)
    loss = lsum / jnp.maximum(lcnt, 1.0)
    return (
        loss.astype(linking_scores.dtype),
        maxs,
        pred.astype(candidate_entities.dtype),
        maski.astype(jnp.bool_),
    )
